# Initial kernel scaffold; baseline (speedup 1.0000x reference)
import jax
import jax.numpy as jnp
from jax.experimental import pallas as pl


def kernel(img_features, vertex_position, vertex_padded, edge_index, w1_loop, w1_neigh, b1, w2_loop, w2_neigh, b2, w3_loop, w3_neigh, b3):
    raise NotImplementedError("write your pallas kernel here")


# speedup vs baseline: 6.0718x; 6.0718x over previous
import functools

import jax
import jax.numpy as jnp
from jax import lax
from jax.experimental import pallas as pl
from jax.experimental.pallas import tpu as pltpu
from jax.experimental.pallas import tpu_sc as plsc

N = 10000
E = 320000
D = 128
DH = D // 2
HH = 56
WW = 56
HW = HH * WW

NC = 2
NS = 16
NWK = NC * NS

V_PAD = 10240
VB = V_PAD // NWK
VC = 80
NVC = VB // VC

ECH = 128
EPS = 20480
NECH = EPS // ECH
E_PAD = EPS * NS

AGG_ROWS = V_PAD


def _mesh():
    return plsc.VectorSubcoreMesh(
        core_axis_name="c", subcore_axis_name="s",
        num_cores=NC, num_subcores=NS)


def _tap_body(xs_ref, ys_ref, idx_ref, w_ref):
    x = xs_ref[...]
    y = ys_ref[...]
    fx = (x + 1.0) * 0.5 * (WW - 1)
    fy = (y + 1.0) * 0.5 * (HH - 1)
    x0 = jnp.floor(fx)
    y0 = jnp.floor(fy)
    x1 = x0 + 1.0
    y1 = y0 + 1.0
    wx1 = fx - x0
    wx0 = 1.0 - wx1
    wy1 = fy - y0
    wy0 = 1.0 - wy1
    taps = ((x0, y0, wx0 * wy0), (x1, y0, wx1 * wy0),
            (x0, y1, wx0 * wy1), (x1, y1, wx1 * wy1))
    for t, (ix, iy, w) in enumerate(taps):
        valid = ((ix >= 0.0) & (ix <= WW - 1.0)
                 & (iy >= 0.0) & (iy <= HH - 1.0))
        ixc = jnp.clip(ix, 0.0, WW - 1.0)
        iyc = jnp.clip(iy, 0.0, HH - 1.0)
        idx_ref[t] = (iyc * WW + ixc).astype(jnp.int32)
        w_ref[t] = jnp.where(valid, w, 0.0)


def _tap_call(xs2, ys2):
    return pl.pallas_call(
        _tap_body,
        out_shape=[
            jax.ShapeDtypeStruct((4, V_PAD // 128, 128), jnp.int32),
            jax.ShapeDtypeStruct((4, V_PAD // 128, 128), jnp.float32),
        ],
    )(xs2, ys2)


def _vert_gather_body(imgt, tidx, taps_out, idxv, rows, sem):
    c = lax.axis_index("c")
    s = lax.axis_index("s")
    wid = c * NS + s
    base = wid * VB
    pltpu.sync_copy(tidx.at[wid], idxv)
    for t in range(4):
        for k in range(NVC):
            cp = pltpu.async_copy(
                imgt.at[idxv.at[pl.ds((t * NVC + k) * VC, VC)]], rows, sem)
            cp.wait()
            pltpu.sync_copy(rows, taps_out.at[t, pl.ds(base + k * VC, VC)])


def _vert_gather(imgt, tidx_w):
    f = functools.partial(
        pl.kernel,
        out_type=jax.ShapeDtypeStruct((4, V_PAD, D), jnp.float32),
        mesh=_mesh(),
        scratch_types=[
            pltpu.VMEM((4 * VB,), jnp.int32),
            pltpu.VMEM((VC, D), jnp.float32),
            pltpu.SemaphoreType.DMA,
        ],
    )(_vert_gather_body)
    return f(imgt, tidx_w)


def _mix_body(taps_ref, w_ref, vpad_ref, wl_ref, b_ref,
              xa_ref, xb_ref, p_ref):
    x0 = vpad_ref[...]
    for t in range(4):
        x0 = x0 + taps_ref[t] * w_ref[t][:, None]
    xa_ref[...] = x0[:, :DH]
    xb_ref[...] = x0[:, DH:]
    p_ref[...] = jnp.dot(x0, wl_ref[...],
                         preferred_element_type=jnp.float32) + b_ref[...]


def _mix_call(taps, w4, vpad, w1l, b1):
    grid = (V_PAD // 256,)
    return pl.pallas_call(
        _mix_body,
        grid=grid,
        in_specs=[
            pl.BlockSpec((4, 256, D), lambda i: (0, i, 0)),
            pl.BlockSpec((4, 256), lambda i: (0, i)),
            pl.BlockSpec((256, D), lambda i: (i, 0)),
            pl.BlockSpec((D, D), lambda i: (0, 0)),
            pl.BlockSpec((1, D), lambda i: (0, 0)),
        ],
        out_specs=[
            pl.BlockSpec((256, DH), lambda i: (i, 0)),
            pl.BlockSpec((256, DH), lambda i: (i, 0)),
            pl.BlockSpec((256, D), lambda i: (i, 0)),
        ],
        out_shape=[
            jax.ShapeDtypeStruct((V_PAD, DH), jnp.float32),
            jax.ShapeDtypeStruct((V_PAD, DH), jnp.float32),
            jax.ShapeDtypeStruct((V_PAD, D), jnp.float32),
        ],
    )(taps, w4, vpad, w1l, b1)


def _agg_body(xa_hbm, xb_hbm, srcr, dstr, zeros_hbm, agg_out,
              srcv, dstv, rb0, rb1, zbuf, aggsh, sem0, sem1):
    c = lax.axis_index("c")
    s = lax.axis_index("s")

    pltpu.sync_copy(zeros_hbm, zbuf)
    rows_per_sub = AGG_ROWS // NS
    for k in range(rows_per_sub // 128):
        pltpu.sync_copy(zbuf, aggsh.at[pl.ds(s * rows_per_sub + k * 128, 128)])
    plsc.subcore_barrier()

    pltpu.sync_copy(srcr.at[s], srcv)
    pltpu.sync_copy(dstr.at[s], dstv)

    def _edge_loop(x_hbm):
        @pl.loop(0, NECH // 2)
        def _chunks(i):
            j = i * 2
            d0 = pltpu.async_copy(
                x_hbm.at[srcv.at[pl.ds(j * ECH, ECH)]], rb0, sem0)
            d1 = pltpu.async_copy(
                x_hbm.at[srcv.at[pl.ds((j + 1) * ECH, ECH)]], rb1, sem1)
            d0.wait()
            pltpu.sync_copy(rb0, aggsh.at[dstv.at[pl.ds(j * ECH, ECH)]],
                            add=True)
            d1.wait()
            pltpu.sync_copy(rb1, aggsh.at[dstv.at[pl.ds((j + 1) * ECH, ECH)]],
                            add=True)

    @pl.when(c == 0)
    def _():
        _edge_loop(xa_hbm)

    @pl.when(c == 1)
    def _():
        _edge_loop(xb_hbm)

    plsc.subcore_barrier()

    for k in range(rows_per_sub // 128):
        off = s * rows_per_sub + k * 128
        pltpu.sync_copy(aggsh.at[pl.ds(off, 128)], rb0)
        pltpu.sync_copy(rb0, agg_out.at[c, pl.ds(off, 128)])


def _agg_call(xa, xb, srcr, dstr, zeros64):
    f = functools.partial(
        pl.kernel,
        out_type=jax.ShapeDtypeStruct((NC, AGG_ROWS, DH), jnp.float32),
        mesh=_mesh(),
        scratch_types=[
            pltpu.VMEM((EPS,), jnp.int32),
            pltpu.VMEM((EPS,), jnp.int32),
            pltpu.VMEM((ECH, DH), jnp.float32),
            pltpu.VMEM((ECH, DH), jnp.float32),
            pltpu.VMEM((128, DH), jnp.float32),
            pltpu.VMEM_SHARED((AGG_ROWS, DH), jnp.float32),
            pltpu.SemaphoreType.DMA,
            pltpu.SemaphoreType.DMA,
        ],
        compiler_params=pltpu.CompilerParams(use_tc_tiling_on_sc=False),
    )(_agg_body)
    return f(xa, xb, srcr, dstr, zeros64)


def _combine_body(p_ref, agg_ref, wn_ref, wl_ref, b_ref,
                  ya_ref, yb_ref, pn_ref):
    agg = jnp.concatenate([agg_ref[0], agg_ref[1]], axis=1)
    y = jnp.maximum(
        p_ref[...] + jnp.dot(agg, wn_ref[...],
                             preferred_element_type=jnp.float32), 0.0)
    ya_ref[...] = y[:, :DH]
    yb_ref[...] = y[:, DH:]
    pn_ref[...] = jnp.dot(y, wl_ref[...],
                          preferred_element_type=jnp.float32) + b_ref[...]


def _combine_call(p, agg, wn, wl_next, b_next):
    grid = (V_PAD // 256,)
    return pl.pallas_call(
        _combine_body,
        grid=grid,
        in_specs=[
            pl.BlockSpec((256, D), lambda i: (i, 0)),
            pl.BlockSpec((NC, 256, DH), lambda i: (0, i, 0)),
            pl.BlockSpec((D, D), lambda i: (0, 0)),
            pl.BlockSpec((D, D), lambda i: (0, 0)),
            pl.BlockSpec((1, D), lambda i: (0, 0)),
        ],
        out_specs=[
            pl.BlockSpec((256, DH), lambda i: (i, 0)),
            pl.BlockSpec((256, DH), lambda i: (i, 0)),
            pl.BlockSpec((256, D), lambda i: (i, 0)),
        ],
        out_shape=[
            jax.ShapeDtypeStruct((V_PAD, DH), jnp.float32),
            jax.ShapeDtypeStruct((V_PAD, DH), jnp.float32),
            jax.ShapeDtypeStruct((V_PAD, D), jnp.float32),
        ],
    )(p, agg, wn, wl_next, b_next)


def _final_body(p_ref, agg_ref, wn_ref, y1a_ref, y1b_ref, out_ref):
    agg = jnp.concatenate([agg_ref[0], agg_ref[1]], axis=1)
    y3 = jnp.maximum(
        p_ref[...] + jnp.dot(agg, wn_ref[...],
                             preferred_element_type=jnp.float32), 0.0)
    y1 = jnp.concatenate([y1a_ref[...], y1b_ref[...]], axis=1)
    out_ref[...] = y1 + y3


def _final_call(p, agg, wn, y1a, y1b):
    grid = (V_PAD // 256,)
    return pl.pallas_call(
        _final_body,
        grid=grid,
        in_specs=[
            pl.BlockSpec((256, D), lambda i: (i, 0)),
            pl.BlockSpec((NC, 256, DH), lambda i: (0, i, 0)),
            pl.BlockSpec((D, D), lambda i: (0, 0)),
            pl.BlockSpec((256, DH), lambda i: (i, 0)),
            pl.BlockSpec((256, DH), lambda i: (i, 0)),
        ],
        out_specs=pl.BlockSpec((256, D), lambda i: (i, 0)),
        out_shape=jax.ShapeDtypeStruct((V_PAD, D), jnp.float32),
    )(p, agg, wn, y1a, y1b)


def kernel(img_features, vertex_position, vertex_padded, edge_index,
           w1_loop, w1_neigh, b1, w2_loop, w2_neigh, b2,
           w3_loop, w3_neigh, b3):
    f32 = jnp.float32

    imgt = img_features.reshape(D, HW).T
    xs = jnp.pad(vertex_position[0, :, 0], (0, V_PAD - N))
    ys = jnp.pad(vertex_position[0, :, 1], (0, V_PAD - N))
    xs2 = xs.reshape(V_PAD // 128, 128)
    ys2 = ys.reshape(V_PAD // 128, 128)
    vpad = jnp.pad(vertex_padded[0], ((0, V_PAD - N), (0, 0)))

    src = edge_index[0]
    dst = edge_index[1]
    pad_n = E_PAD - E
    src_p = jnp.concatenate([src, jnp.arange(pad_n, dtype=jnp.int32) % N])
    dst_p = jnp.concatenate(
        [dst, N + (jnp.arange(pad_n, dtype=jnp.int32) % (AGG_ROWS - N))])
    srcr = src_p.reshape(NS, EPS)
    dstr = dst_p.reshape(NS, EPS)
    zeros64 = jnp.zeros((128, DH), f32)

    b1r = b1.reshape(1, D)
    b2r = b2.reshape(1, D)
    b3r = b3.reshape(1, D)

    tidx, tw = _tap_call(xs2, ys2)
    tidx_w = tidx.reshape(4, NWK, VB).transpose(1, 0, 2).reshape(NWK, 4 * VB)
    w4 = tw.reshape(4, V_PAD)

    taps = _vert_gather(imgt, tidx_w)

    x0a, x0b, p1 = _mix_call(taps, w4, vpad, w1_loop, b1r)

    agg1 = _agg_call(x0a, x0b, srcr, dstr, zeros64)
    y1a, y1b, p2 = _combine_call(p1, agg1, w1_neigh, w2_loop, b2r)
    agg2 = _agg_call(y1a, y1b, srcr, dstr, zeros64)
    y2a, y2b, p3 = _combine_call(p2, agg2, w2_neigh, w3_loop, b3r)
    agg3 = _agg_call(y2a, y2b, srcr, dstr, zeros64)
    out = _final_call(p3, agg3, w3_neigh, y1a, y1b)

    return out[:N][None, :, :]


# speedup vs baseline: 8.6687x; 1.4277x over previous
import functools

import jax
import jax.numpy as jnp
from jax import lax
from jax.experimental import pallas as pl
from jax.experimental.pallas import tpu as pltpu
from jax.experimental.pallas import tpu_sc as plsc

N = 10000
E = 320000
D = 128
DH = D // 2
HH = 56
WW = 56
HW = HH * WW

NC = 2
NS = 16
NWK = NC * NS

V_PAD = 10240
VB = V_PAD // NWK
VC = 80
NVC = VB // VC

ECH = 128
EPS = 20480
NECH = EPS // ECH
E_PAD = EPS * NS

AGG_ROWS = V_PAD


def _mesh():
    return plsc.VectorSubcoreMesh(
        core_axis_name="c", subcore_axis_name="s",
        num_cores=NC, num_subcores=NS)


def _tap_body(xs_ref, ys_ref, idx_ref, w_ref):
    x = xs_ref[...]
    y = ys_ref[...]
    fx = (x + 1.0) * 0.5 * (WW - 1)
    fy = (y + 1.0) * 0.5 * (HH - 1)
    x0 = jnp.floor(fx)
    y0 = jnp.floor(fy)
    x1 = x0 + 1.0
    y1 = y0 + 1.0
    wx1 = fx - x0
    wx0 = 1.0 - wx1
    wy1 = fy - y0
    wy0 = 1.0 - wy1
    taps = ((x0, y0, wx0 * wy0), (x1, y0, wx1 * wy0),
            (x0, y1, wx0 * wy1), (x1, y1, wx1 * wy1))
    for t, (ix, iy, w) in enumerate(taps):
        valid = ((ix >= 0.0) & (ix <= WW - 1.0)
                 & (iy >= 0.0) & (iy <= HH - 1.0))
        ixc = jnp.clip(ix, 0.0, WW - 1.0)
        iyc = jnp.clip(iy, 0.0, HH - 1.0)
        idx_ref[t] = (iyc * WW + ixc).astype(jnp.int32)
        w_ref[t] = jnp.where(valid, w, 0.0)


def _tap_call(xs2, ys2):
    return pl.pallas_call(
        _tap_body,
        out_shape=[
            jax.ShapeDtypeStruct((4, V_PAD // 128, 128), jnp.int32),
            jax.ShapeDtypeStruct((4, V_PAD // 128, 128), jnp.float32),
        ],
    )(xs2, ys2)


def _vert_gather_body(imgt, tidx, taps_out, idxv, rows0, rows1, sem0, sem1):
    c = lax.axis_index("c")
    s = lax.axis_index("s")
    wid = c * NS + s
    base = wid * VB
    pltpu.sync_copy(tidx.at[wid], idxv)
    rows = (rows0, rows1)
    sems = (sem0, sem1)

    def _issue(n):
        return pltpu.async_copy(
            imgt.at[idxv.at[pl.ds(n * VC, VC)]], rows[n % 2], sems[n % 2])

    n_chunks = 4 * NVC
    d = _issue(0)
    for n in range(n_chunks):
        d_next = _issue(n + 1) if n + 1 < n_chunks else None
        d.wait()
        t, k = divmod(n, NVC)
        pltpu.sync_copy(rows[n % 2],
                        taps_out.at[t, pl.ds(base + k * VC, VC)])
        d = d_next


def _vert_gather(imgt, tidx_w):
    f = functools.partial(
        pl.kernel,
        out_type=jax.ShapeDtypeStruct((4, V_PAD, D), jnp.float32),
        mesh=_mesh(),
        scratch_types=[
            pltpu.VMEM((4 * VB,), jnp.int32),
            pltpu.VMEM((VC, D), jnp.float32),
            pltpu.VMEM((VC, D), jnp.float32),
            pltpu.SemaphoreType.DMA,
            pltpu.SemaphoreType.DMA,
        ],
    )(_vert_gather_body)
    return f(imgt, tidx_w)


def _mix_body(taps_ref, w_ref, vpad_ref, wl_ref, b_ref,
              xa_ref, xb_ref, p_ref):
    x0 = vpad_ref[...]
    for t in range(4):
        x0 = x0 + taps_ref[t] * w_ref[t][:, None]
    xa_ref[...] = x0[:, :DH]
    xb_ref[...] = x0[:, DH:]
    p_ref[...] = jnp.dot(x0, wl_ref[...],
                         preferred_element_type=jnp.float32) + b_ref[...]


def _mix_call(taps, w4, vpad, w1l, b1):
    grid = (V_PAD // 256,)
    return pl.pallas_call(
        _mix_body,
        grid=grid,
        in_specs=[
            pl.BlockSpec((4, 256, D), lambda i: (0, i, 0)),
            pl.BlockSpec((4, 256), lambda i: (0, i)),
            pl.BlockSpec((256, D), lambda i: (i, 0)),
            pl.BlockSpec((D, D), lambda i: (0, 0)),
            pl.BlockSpec((1, D), lambda i: (0, 0)),
        ],
        out_specs=[
            pl.BlockSpec((256, DH), lambda i: (i, 0)),
            pl.BlockSpec((256, DH), lambda i: (i, 0)),
            pl.BlockSpec((256, D), lambda i: (i, 0)),
        ],
        out_shape=[
            jax.ShapeDtypeStruct((V_PAD, DH), jnp.float32),
            jax.ShapeDtypeStruct((V_PAD, DH), jnp.float32),
            jax.ShapeDtypeStruct((V_PAD, D), jnp.float32),
        ],
    )(taps, w4, vpad, w1l, b1)


def _agg_body(xa_hbm, xb_hbm, srcr, dstr, zeros_hbm, agg_out,
              srcv, dstv, rb0, rb1, rb2, rb3, zbuf, aggsh,
              sem0, sem1, sem2, sem3):
    c = lax.axis_index("c")
    s = lax.axis_index("s")

    pltpu.sync_copy(zeros_hbm, zbuf)
    rows_per_sub = AGG_ROWS // NS
    for k in range(rows_per_sub // 128):
        pltpu.sync_copy(zbuf, aggsh.at[pl.ds(s * rows_per_sub + k * 128, 128)])
    plsc.subcore_barrier()

    pltpu.sync_copy(srcr.at[s], srcv)
    pltpu.sync_copy(dstr.at[s], dstv)

    def _edge_loop(x_hbm):
        rbs = (rb0, rb1, rb2, rb3)
        sems = (sem0, sem1, sem2, sem3)

        def _gather(jj, b):
            return pltpu.async_copy(
                x_hbm.at[srcv.at[pl.ds(jj * ECH, ECH)]], rbs[b], sems[b])

        for b in range(4):
            _gather(b, b)

        @pl.loop(0, NECH // 4)
        def _chunks(i):
            j = i * 4
            for b in range(4):
                jj = j + b
                pltpu.make_async_copy(
                    x_hbm.at[srcv.at[pl.ds(jj * ECH, ECH)]],
                    rbs[b], sems[b]).wait()
                pltpu.sync_copy(rbs[b],
                                aggsh.at[dstv.at[pl.ds(jj * ECH, ECH)]],
                                add=True)
                jn = jj + 4

                @pl.when(jn < NECH)
                def _():
                    _gather(jn, b)

    @pl.when(c == 0)
    def _():
        _edge_loop(xa_hbm)

    @pl.when(c == 1)
    def _():
        _edge_loop(xb_hbm)

    plsc.subcore_barrier()

    for k in range(rows_per_sub // 128):
        off = s * rows_per_sub + k * 128
        pltpu.sync_copy(aggsh.at[pl.ds(off, 128)], rb0)
        pltpu.sync_copy(rb0, agg_out.at[c, pl.ds(off, 128)])


def _agg_call(xa, xb, srcr, dstr, zeros64):
    f = functools.partial(
        pl.kernel,
        out_type=jax.ShapeDtypeStruct((NC, AGG_ROWS, DH), jnp.float32),
        mesh=_mesh(),
        scratch_types=[
            pltpu.VMEM((EPS,), jnp.int32),
            pltpu.VMEM((EPS,), jnp.int32),
            pltpu.VMEM((ECH, DH), jnp.float32),
            pltpu.VMEM((ECH, DH), jnp.float32),
            pltpu.VMEM((ECH, DH), jnp.float32),
            pltpu.VMEM((ECH, DH), jnp.float32),
            pltpu.VMEM((128, DH), jnp.float32),
            pltpu.VMEM_SHARED((AGG_ROWS, DH), jnp.float32),
            pltpu.SemaphoreType.DMA,
            pltpu.SemaphoreType.DMA,
            pltpu.SemaphoreType.DMA,
            pltpu.SemaphoreType.DMA,
        ],
        compiler_params=pltpu.CompilerParams(use_tc_tiling_on_sc=False),
    )(_agg_body)
    return f(xa, xb, srcr, dstr, zeros64)


def _combine_body(p_ref, agg_ref, wn_ref, wl_ref, b_ref,
                  ya_ref, yb_ref, pn_ref):
    agg = jnp.concatenate([agg_ref[0], agg_ref[1]], axis=1)
    y = jnp.maximum(
        p_ref[...] + jnp.dot(agg, wn_ref[...],
                             preferred_element_type=jnp.float32), 0.0)
    ya_ref[...] = y[:, :DH]
    yb_ref[...] = y[:, DH:]
    pn_ref[...] = jnp.dot(y, wl_ref[...],
                          preferred_element_type=jnp.float32) + b_ref[...]


def _combine_call(p, agg, wn, wl_next, b_next):
    grid = (V_PAD // 256,)
    return pl.pallas_call(
        _combine_body,
        grid=grid,
        in_specs=[
            pl.BlockSpec((256, D), lambda i: (i, 0)),
            pl.BlockSpec((NC, 256, DH), lambda i: (0, i, 0)),
            pl.BlockSpec((D, D), lambda i: (0, 0)),
            pl.BlockSpec((D, D), lambda i: (0, 0)),
            pl.BlockSpec((1, D), lambda i: (0, 0)),
        ],
        out_specs=[
            pl.BlockSpec((256, DH), lambda i: (i, 0)),
            pl.BlockSpec((256, DH), lambda i: (i, 0)),
            pl.BlockSpec((256, D), lambda i: (i, 0)),
        ],
        out_shape=[
            jax.ShapeDtypeStruct((V_PAD, DH), jnp.float32),
            jax.ShapeDtypeStruct((V_PAD, DH), jnp.float32),
            jax.ShapeDtypeStruct((V_PAD, D), jnp.float32),
        ],
    )(p, agg, wn, wl_next, b_next)


def _final_body(p_ref, agg_ref, wn_ref, y1a_ref, y1b_ref, out_ref):
    agg = jnp.concatenate([agg_ref[0], agg_ref[1]], axis=1)
    y3 = jnp.maximum(
        p_ref[...] + jnp.dot(agg, wn_ref[...],
                             preferred_element_type=jnp.float32), 0.0)
    y1 = jnp.concatenate([y1a_ref[...], y1b_ref[...]], axis=1)
    out_ref[...] = y1 + y3


def _final_call(p, agg, wn, y1a, y1b):
    grid = (V_PAD // 256,)
    return pl.pallas_call(
        _final_body,
        grid=grid,
        in_specs=[
            pl.BlockSpec((256, D), lambda i: (i, 0)),
            pl.BlockSpec((NC, 256, DH), lambda i: (0, i, 0)),
            pl.BlockSpec((D, D), lambda i: (0, 0)),
            pl.BlockSpec((256, DH), lambda i: (i, 0)),
            pl.BlockSpec((256, DH), lambda i: (i, 0)),
        ],
        out_specs=pl.BlockSpec((256, D), lambda i: (i, 0)),
        out_shape=jax.ShapeDtypeStruct((V_PAD, D), jnp.float32),
    )(p, agg, wn, y1a, y1b)


def kernel(img_features, vertex_position, vertex_padded, edge_index,
           w1_loop, w1_neigh, b1, w2_loop, w2_neigh, b2,
           w3_loop, w3_neigh, b3):
    f32 = jnp.float32

    imgt = img_features.reshape(D, HW).T
    xs = jnp.pad(vertex_position[0, :, 0], (0, V_PAD - N))
    ys = jnp.pad(vertex_position[0, :, 1], (0, V_PAD - N))
    xs2 = xs.reshape(V_PAD // 128, 128)
    ys2 = ys.reshape(V_PAD // 128, 128)
    vpad = jnp.pad(vertex_padded[0], ((0, V_PAD - N), (0, 0)))

    src = edge_index[0]
    dst = edge_index[1]
    pad_n = E_PAD - E
    src_p = jnp.concatenate([src, jnp.arange(pad_n, dtype=jnp.int32) % N])
    dst_p = jnp.concatenate(
        [dst, N + (jnp.arange(pad_n, dtype=jnp.int32) % (AGG_ROWS - N))])
    srcr = src_p.reshape(NS, EPS)
    dstr = dst_p.reshape(NS, EPS)
    zeros64 = jnp.zeros((128, DH), f32)

    b1r = b1.reshape(1, D)
    b2r = b2.reshape(1, D)
    b3r = b3.reshape(1, D)

    tidx, tw = _tap_call(xs2, ys2)
    tidx_w = tidx.reshape(4, NWK, VB).transpose(1, 0, 2).reshape(NWK, 4 * VB)
    w4 = tw.reshape(4, V_PAD)

    taps = _vert_gather(imgt, tidx_w)

    x0a, x0b, p1 = _mix_call(taps, w4, vpad, w1_loop, b1r)

    agg1 = _agg_call(x0a, x0b, srcr, dstr, zeros64)
    y1a, y1b, p2 = _combine_call(p1, agg1, w1_neigh, w2_loop, b2r)
    agg2 = _agg_call(y1a, y1b, srcr, dstr, zeros64)
    y2a, y2b, p3 = _combine_call(p2, agg2, w2_neigh, w3_loop, b3r)
    agg3 = _agg_call(y2a, y2b, srcr, dstr, zeros64)
    out = _final_call(p3, agg3, w3_neigh, y1a, y1b)

    return out[:N][None, :, :]


# speedup vs baseline: 10.8742x; 1.2544x over previous
import functools

import jax
import jax.numpy as jnp
from jax import lax
from jax.experimental import pallas as pl
from jax.experimental.pallas import tpu as pltpu
from jax.experimental.pallas import tpu_sc as plsc

N = 10000
E = 320000
D = 128
DH = D // 2
HH = 56
WW = 56
HW = HH * WW

NC = 2
NS = 16
NWK = NC * NS

V_PAD = 10240
VB = V_PAD // NWK
VC = 80
NVC = VB // VC

ECH = 128
EPS = 20480
NECH = EPS // ECH
E_PAD = EPS * NS

AGG_ROWS = V_PAD


def _mesh():
    return plsc.VectorSubcoreMesh(
        core_axis_name="c", subcore_axis_name="s",
        num_cores=NC, num_subcores=NS)


def _tap_body(xs_ref, ys_ref, idx_ref, w_ref):
    x = xs_ref[...]
    y = ys_ref[...]
    fx = (x + 1.0) * 0.5 * (WW - 1)
    fy = (y + 1.0) * 0.5 * (HH - 1)
    x0 = jnp.floor(fx)
    y0 = jnp.floor(fy)
    x1 = x0 + 1.0
    y1 = y0 + 1.0
    wx1 = fx - x0
    wx0 = 1.0 - wx1
    wy1 = fy - y0
    wy0 = 1.0 - wy1
    taps = ((x0, y0, wx0 * wy0), (x1, y0, wx1 * wy0),
            (x0, y1, wx0 * wy1), (x1, y1, wx1 * wy1))
    for t, (ix, iy, w) in enumerate(taps):
        valid = ((ix >= 0.0) & (ix <= WW - 1.0)
                 & (iy >= 0.0) & (iy <= HH - 1.0))
        ixc = jnp.clip(ix, 0.0, WW - 1.0)
        iyc = jnp.clip(iy, 0.0, HH - 1.0)
        idx_ref[t] = (iyc * WW + ixc).astype(jnp.int32)
        w_ref[t] = jnp.where(valid, w, 0.0)


def _tap_call(xs2, ys2):
    return pl.pallas_call(
        _tap_body,
        out_shape=[
            jax.ShapeDtypeStruct((4, V_PAD // 128, 128), jnp.int32),
            jax.ShapeDtypeStruct((4, V_PAD // 128, 128), jnp.float32),
        ],
    )(xs2, ys2)


def _vert_gather_body(imgt, tidx, taps_out, idxv, rows0, rows1, sem0, sem1):
    c = lax.axis_index("c")
    s = lax.axis_index("s")
    wid = c * NS + s
    base = wid * VB
    pltpu.sync_copy(tidx.at[wid], idxv)
    rows = (rows0, rows1)
    sems = (sem0, sem1)

    def _issue(n):
        return pltpu.async_copy(
            imgt.at[idxv.at[pl.ds(n * VC, VC)]], rows[n % 2], sems[n % 2])

    n_chunks = 4 * NVC
    d = _issue(0)
    for n in range(n_chunks):
        d_next = _issue(n + 1) if n + 1 < n_chunks else None
        d.wait()
        t, k = divmod(n, NVC)
        pltpu.sync_copy(rows[n % 2],
                        taps_out.at[t, pl.ds(base + k * VC, VC)])
        d = d_next


def _vert_gather(imgt, tidx_w):
    f = functools.partial(
        pl.kernel,
        out_type=jax.ShapeDtypeStruct((4, V_PAD, D), jnp.float32),
        mesh=_mesh(),
        scratch_types=[
            pltpu.VMEM((4 * VB,), jnp.int32),
            pltpu.VMEM((VC, D), jnp.float32),
            pltpu.VMEM((VC, D), jnp.float32),
            pltpu.SemaphoreType.DMA,
            pltpu.SemaphoreType.DMA,
        ],
    )(_vert_gather_body)
    return f(imgt, tidx_w)


def _mix_body(taps_ref, w_ref, vpad_ref, wl_ref, b_ref,
              xa_ref, xb_ref, p_ref):
    x0 = vpad_ref[...]
    for t in range(4):
        x0 = x0 + taps_ref[t] * w_ref[t][:, None]
    xa_ref[...] = x0[:, :DH]
    xb_ref[...] = x0[:, DH:]
    p_ref[...] = jnp.dot(x0, wl_ref[...],
                         preferred_element_type=jnp.float32) + b_ref[...]


def _mix_call(taps, w4, vpad, w1l, b1):
    R = 1024
    grid = (V_PAD // R,)
    return pl.pallas_call(
        _mix_body,
        grid=grid,
        in_specs=[
            pl.BlockSpec((4, R, D), lambda i: (0, i, 0)),
            pl.BlockSpec((4, R), lambda i: (0, i)),
            pl.BlockSpec((R, D), lambda i: (i, 0)),
            pl.BlockSpec((D, D), lambda i: (0, 0)),
            pl.BlockSpec((1, D), lambda i: (0, 0)),
        ],
        out_specs=[
            pl.BlockSpec((R, DH), lambda i: (i, 0)),
            pl.BlockSpec((R, DH), lambda i: (i, 0)),
            pl.BlockSpec((R, D), lambda i: (i, 0)),
        ],
        compiler_params=pltpu.CompilerParams(
            dimension_semantics=("arbitrary",)),
        out_shape=[
            jax.ShapeDtypeStruct((V_PAD, DH), jnp.float32),
            jax.ShapeDtypeStruct((V_PAD, DH), jnp.float32),
            jax.ShapeDtypeStruct((V_PAD, D), jnp.float32),
        ],
    )(taps, w4, vpad, w1l, b1)


def _agg_body(xa_hbm, xb_hbm, srcr, dstr, zeros_hbm, agg_out,
              srcv, dstv, rb0, rb1, rb2, rb3, zbuf, aggsh,
              sem0, sem1, sem2, sem3):
    c = lax.axis_index("c")
    s = lax.axis_index("s")

    pltpu.sync_copy(zeros_hbm, zbuf)
    rows_per_sub = AGG_ROWS // NS
    for k in range(rows_per_sub // 128):
        pltpu.sync_copy(zbuf, aggsh.at[pl.ds(s * rows_per_sub + k * 128, 128)])
    plsc.subcore_barrier()

    pltpu.sync_copy(srcr.at[s], srcv)
    pltpu.sync_copy(dstr.at[s], dstv)

    def _edge_loop(x_hbm):
        rbs = (rb0, rb1, rb2, rb3)
        sems = (sem0, sem1, sem2, sem3)

        def _gather(jj, b):
            return pltpu.async_copy(
                x_hbm.at[srcv.at[pl.ds(jj * ECH, ECH)]], rbs[b], sems[b])

        for b in range(4):
            _gather(b, b)

        @pl.loop(0, NECH // 4)
        def _chunks(i):
            j = i * 4
            for b in range(4):
                jj = j + b
                pltpu.make_async_copy(
                    x_hbm.at[srcv.at[pl.ds(jj * ECH, ECH)]],
                    rbs[b], sems[b]).wait()
                pltpu.sync_copy(rbs[b],
                                aggsh.at[dstv.at[pl.ds(jj * ECH, ECH)]],
                                add=True)
                jn = jj + 4

                @pl.when(jn < NECH)
                def _():
                    _gather(jn, b)

    @pl.when(c == 0)
    def _():
        _edge_loop(xa_hbm)

    @pl.when(c == 1)
    def _():
        _edge_loop(xb_hbm)

    plsc.subcore_barrier()

    for k in range(rows_per_sub // 128):
        off = s * rows_per_sub + k * 128
        pltpu.sync_copy(aggsh.at[pl.ds(off, 128)], rb0)
        pltpu.sync_copy(rb0, agg_out.at[c, pl.ds(off, 128)])


def _agg_call(xa, xb, srcr, dstr, zeros64):
    f = functools.partial(
        pl.kernel,
        out_type=jax.ShapeDtypeStruct((NC, AGG_ROWS, DH), jnp.float32),
        mesh=_mesh(),
        scratch_types=[
            pltpu.VMEM((EPS,), jnp.int32),
            pltpu.VMEM((EPS,), jnp.int32),
            pltpu.VMEM((ECH, DH), jnp.float32),
            pltpu.VMEM((ECH, DH), jnp.float32),
            pltpu.VMEM((ECH, DH), jnp.float32),
            pltpu.VMEM((ECH, DH), jnp.float32),
            pltpu.VMEM((128, DH), jnp.float32),
            pltpu.VMEM_SHARED((AGG_ROWS, DH), jnp.float32),
            pltpu.SemaphoreType.DMA,
            pltpu.SemaphoreType.DMA,
            pltpu.SemaphoreType.DMA,
            pltpu.SemaphoreType.DMA,
        ],
        compiler_params=pltpu.CompilerParams(use_tc_tiling_on_sc=False),
    )(_agg_body)
    return f(xa, xb, srcr, dstr, zeros64)


def _combine_body(p_ref, agg_ref, wn_ref, wl_ref, b_ref,
                  ya_ref, yb_ref, pn_ref):
    agg = jnp.concatenate([agg_ref[0], agg_ref[1]], axis=1)
    y = jnp.maximum(
        p_ref[...] + jnp.dot(agg, wn_ref[...],
                             preferred_element_type=jnp.float32), 0.0)
    ya_ref[...] = y[:, :DH]
    yb_ref[...] = y[:, DH:]
    pn_ref[...] = jnp.dot(y, wl_ref[...],
                          preferred_element_type=jnp.float32) + b_ref[...]


def _combine_call(p, agg, wn, wl_next, b_next):
    R = 1024
    grid = (V_PAD // R,)
    return pl.pallas_call(
        _combine_body,
        grid=grid,
        in_specs=[
            pl.BlockSpec((R, D), lambda i: (i, 0)),
            pl.BlockSpec((NC, R, DH), lambda i: (0, i, 0)),
            pl.BlockSpec((D, D), lambda i: (0, 0)),
            pl.BlockSpec((D, D), lambda i: (0, 0)),
            pl.BlockSpec((1, D), lambda i: (0, 0)),
        ],
        out_specs=[
            pl.BlockSpec((R, DH), lambda i: (i, 0)),
            pl.BlockSpec((R, DH), lambda i: (i, 0)),
            pl.BlockSpec((R, D), lambda i: (i, 0)),
        ],
        compiler_params=pltpu.CompilerParams(
            dimension_semantics=("arbitrary",)),
        out_shape=[
            jax.ShapeDtypeStruct((V_PAD, DH), jnp.float32),
            jax.ShapeDtypeStruct((V_PAD, DH), jnp.float32),
            jax.ShapeDtypeStruct((V_PAD, D), jnp.float32),
        ],
    )(p, agg, wn, wl_next, b_next)


def _final_body(p_ref, agg_ref, wn_ref, y1a_ref, y1b_ref, out_ref):
    agg = jnp.concatenate([agg_ref[0], agg_ref[1]], axis=1)
    y3 = jnp.maximum(
        p_ref[...] + jnp.dot(agg, wn_ref[...],
                             preferred_element_type=jnp.float32), 0.0)
    y1 = jnp.concatenate([y1a_ref[...], y1b_ref[...]], axis=1)
    out_ref[...] = y1 + y3


def _final_call(p, agg, wn, y1a, y1b):
    R = 1024
    grid = (V_PAD // R,)
    return pl.pallas_call(
        _final_body,
        grid=grid,
        in_specs=[
            pl.BlockSpec((R, D), lambda i: (i, 0)),
            pl.BlockSpec((NC, R, DH), lambda i: (0, i, 0)),
            pl.BlockSpec((D, D), lambda i: (0, 0)),
            pl.BlockSpec((R, DH), lambda i: (i, 0)),
            pl.BlockSpec((R, DH), lambda i: (i, 0)),
        ],
        out_specs=pl.BlockSpec((R, D), lambda i: (i, 0)),
        out_shape=jax.ShapeDtypeStruct((V_PAD, D), jnp.float32),
        compiler_params=pltpu.CompilerParams(
            dimension_semantics=("arbitrary",)),
    )(p, agg, wn, y1a, y1b)


def kernel(img_features, vertex_position, vertex_padded, edge_index,
           w1_loop, w1_neigh, b1, w2_loop, w2_neigh, b2,
           w3_loop, w3_neigh, b3):
    f32 = jnp.float32

    imgt = img_features.reshape(D, HW).T
    pad_coord = jnp.linspace(-0.95, 0.95, V_PAD - N, dtype=f32)
    xs = jnp.concatenate([vertex_position[0, :, 0], pad_coord])
    ys = jnp.concatenate([vertex_position[0, :, 1], -pad_coord])
    xs2 = xs.reshape(V_PAD // 128, 128)
    ys2 = ys.reshape(V_PAD // 128, 128)
    vpad = jnp.pad(vertex_padded[0], ((0, V_PAD - N), (0, 0)))

    src = edge_index[0]
    dst = edge_index[1]
    pad_n = E_PAD - E
    src_p = jnp.concatenate([src, jnp.arange(pad_n, dtype=jnp.int32) % N])
    dst_p = jnp.concatenate(
        [dst, N + (jnp.arange(pad_n, dtype=jnp.int32) % (AGG_ROWS - N))])
    srcr = src_p.reshape(NS, EPS)
    dstr = dst_p.reshape(NS, EPS)
    zeros64 = jnp.zeros((128, DH), f32)

    b1r = b1.reshape(1, D)
    b2r = b2.reshape(1, D)
    b3r = b3.reshape(1, D)

    tidx, tw = _tap_call(xs2, ys2)
    tidx_w = tidx.reshape(4, NWK, VB).transpose(1, 0, 2).reshape(NWK, 4 * VB)
    w4 = tw.reshape(4, V_PAD)

    taps = _vert_gather(imgt, tidx_w)

    x0a, x0b, p1 = _mix_call(taps, w4, vpad, w1_loop, b1r)

    agg1 = _agg_call(x0a, x0b, srcr, dstr, zeros64)
    y1a, y1b, p2 = _combine_call(p1, agg1, w1_neigh, w2_loop, b2r)
    agg2 = _agg_call(y1a, y1b, srcr, dstr, zeros64)
    y2a, y2b, p3 = _combine_call(p2, agg2, w2_neigh, w3_loop, b3r)
    agg3 = _agg_call(y2a, y2b, srcr, dstr, zeros64)
    out = _final_call(p3, agg3, w3_neigh, y1a, y1b)

    return out[:N][None, :, :]
